# baseline (speedup 1.0000x reference)
import functools

import jax
import jax.numpy as jnp
from jax import lax
from jax.experimental import pallas as pl
from jax.experimental.pallas import tpu as pltpu
from jax.experimental.pallas import tpu_sc as plsc

F = 32
NC = 2
NS = 16
NW = NC * NS
CH = 128


def _gather4_sc(user2d, item2d, t_ug, t_ig, t_um, t_im, B):
    bpw = B // NW
    nch = bpw // CH
    mesh = plsc.VectorSubcoreMesh(core_axis_name="c", subcore_axis_name="s")
    out_t = [jax.ShapeDtypeStruct((B, F), jnp.float32)] * 4

    @functools.partial(
        pl.kernel,
        out_type=out_t,
        mesh=mesh,
        compiler_params=pltpu.CompilerParams(use_tc_tiling_on_sc=False),
        scratch_types=[
            pltpu.VMEM((nch, CH), jnp.int32),
            pltpu.VMEM((nch, CH), jnp.int32),
            pltpu.VMEM((bpw, F), jnp.float32),
            pltpu.VMEM((bpw, F), jnp.float32),
            pltpu.VMEM((bpw, F), jnp.float32),
            pltpu.VMEM((bpw, F), jnp.float32),
            pltpu.SemaphoreType.DMA,
        ],
    )
    def gather_kernel(u_hbm, i_hbm, tug, tig, tum, tim,
                      o_ug, o_ig, o_um, o_im,
                      idx_u, idx_i, r_ug, r_ig, r_um, r_im, sem):
        wid = lax.axis_index("s") * NC + lax.axis_index("c")
        rowblk = wid * nch
        pltpu.sync_copy(u_hbm.at[pl.ds(rowblk, nch)], idx_u)
        pltpu.sync_copy(i_hbm.at[pl.ds(rowblk, nch)], idx_i)
        descs = []
        for j in range(nch):
            dst = pl.ds(j * CH, CH)
            descs.append(pltpu.async_copy(tug.at[idx_u.at[j]], r_ug.at[dst], sem))
            descs.append(pltpu.async_copy(tig.at[idx_i.at[j]], r_ig.at[dst], sem))
            descs.append(pltpu.async_copy(tum.at[idx_u.at[j]], r_um.at[dst], sem))
            descs.append(pltpu.async_copy(tim.at[idx_i.at[j]], r_im.at[dst], sem))
        for d in descs:
            d.wait()
        base = wid * bpw
        pltpu.sync_copy(r_ug, o_ug.at[pl.ds(base, bpw)])
        pltpu.sync_copy(r_ig, o_ig.at[pl.ds(base, bpw)])
        pltpu.sync_copy(r_um, o_um.at[pl.ds(base, bpw)])
        pltpu.sync_copy(r_im, o_im.at[pl.ds(base, bpw)])

    return gather_kernel(user2d, item2d, t_ug, t_ig, t_um, t_im)


def _mish(x):
    return x * jnp.tanh(jax.nn.softplus(x))


def _mlp_body(eug, eig, eum, eim, w0a, w0b, b0r, w1, b1r, w2, b2r,
              wpa, wpb, bpr, out):
    h = jnp.dot(eum[...], w0a[...]) + jnp.dot(eim[...], w0b[...]) + b0r[...]
    h = _mish(h)
    h = _mish(jnp.dot(h, w1[...]) + b1r[...])
    h = _mish(jnp.dot(h, w2[...]) + b2r[...])
    g = eug[...] * eig[...]
    p = (jnp.sum(g * wpa[...], axis=1, keepdims=True)
         + jnp.sum(h * wpb[...], axis=1, keepdims=True) + bpr[...])
    out[...] = _mish(p)


def _mlp_tc(eu_g, ei_g, eu_m, ei_m, W0, b0, W1, b1, W2, b2, Wp, bp, B):
    blk = 2048
    grid = (B // blk,)
    w0a = W0[:F]
    w0b = W0[F:]
    wpa = Wp[:F].reshape(1, F)
    wpb = Wp[F:].reshape(1, F)
    b0r = b0.reshape(1, -1)
    b1r = b1.reshape(1, -1)
    b2r = b2.reshape(1, -1)
    bpr = bp.reshape(1, 1)

    def row_spec(d):
        return pl.BlockSpec((blk, d), lambda i: (i, 0))

    def full_spec(a):
        return pl.BlockSpec(a.shape, lambda i: (0,) * a.ndim)

    out = pl.pallas_call(
        _mlp_body,
        grid=grid,
        in_specs=[
            row_spec(F), row_spec(F), row_spec(F), row_spec(F),
            full_spec(w0a), full_spec(w0b), full_spec(b0r),
            full_spec(W1), full_spec(b1r),
            full_spec(W2), full_spec(b2r),
            full_spec(wpa), full_spec(wpb), full_spec(bpr),
        ],
        out_specs=pl.BlockSpec((blk, 1), lambda i: (i, 0)),
        out_shape=jax.ShapeDtypeStruct((B, 1), jnp.float32),
    )(eu_g, ei_g, eu_m, ei_m, w0a, w0b, b0r, W1, b1r, W2, b2r, wpa, wpb, bpr)
    return out.reshape(-1)


def kernel(user, item, embed_user_GMF, embed_item_GMF, embed_user_MLP,
           embed_item_MLP, W0, b0, W1, b1, W2, b2, Wp, bp):
    B = user.shape[0]
    u2 = user.astype(jnp.int32).reshape(B // CH, CH)
    i2 = item.astype(jnp.int32).reshape(B // CH, CH)
    eu_g, ei_g, eu_m, ei_m = _gather4_sc(
        u2, i2, embed_user_GMF, embed_item_GMF, embed_user_MLP,
        embed_item_MLP, B)
    return _mlp_tc(eu_g, ei_g, eu_m, ei_m, W0, b0, W1, b1, W2, b2, Wp, bp, B)
